# Initial kernel scaffold; baseline (speedup 1.0000x reference)
import jax
import jax.numpy as jnp
from jax.experimental import pallas as pl


def kernel(X, H, Y, Wy, By, W_ih, W_hh, b_ih, b_hh):
    raise NotImplementedError("write your pallas kernel here")


# speedup vs baseline: 2.2353x; 2.2353x over previous
import functools

import jax
import jax.numpy as jnp
from jax import lax
from jax.experimental import pallas as pl
from jax.experimental.pallas import tpu as pltpu
from jax.experimental.pallas import tpu_sc as plsc


def _sc_gather(Wy, By_flat, X, Y):
    B = X.shape[0]
    D = Wy.shape[1]
    NW = 32
    CH = B // NW

    mesh = plsc.VectorSubcoreMesh(core_axis_name="c", subcore_axis_name="s")

    @functools.partial(
        pl.kernel,
        out_type=(
            jax.ShapeDtypeStruct((B, D), jnp.float32),
            jax.ShapeDtypeStruct((B, D), jnp.float32),
            jax.ShapeDtypeStruct((B,), jnp.float32),
        ),
        mesh=mesh,
        scratch_types=[
            pltpu.VMEM((2, CH), jnp.int32),
            pltpu.VMEM((CH, D), jnp.float32),
            pltpu.VMEM((CH, D), jnp.float32),
            pltpu.VMEM((CH,), jnp.float32),
            pltpu.SemaphoreType.DMA,
        ],
    )
    def k(wy_hbm, by_hbm, x_hbm, y_hbm, e_hbm, o_hbm, bb_hbm,
          idx_v, ex_v, oy_v, bv_v, sem):
        wid = lax.axis_index("s") * 2 + lax.axis_index("c")
        base = wid * CH
        pltpu.sync_copy(x_hbm.at[pl.ds(base, CH)], idx_v.at[0])
        pltpu.sync_copy(y_hbm.at[pl.ds(base, CH)], idx_v.at[1])
        c1 = pltpu.async_copy(wy_hbm.at[idx_v.at[0]], ex_v, sem)
        c2 = pltpu.async_copy(wy_hbm.at[idx_v.at[1]], oy_v, sem)
        c3 = pltpu.async_copy(by_hbm.at[idx_v.at[1]], bv_v, sem)
        c1.wait()
        c2.wait()
        c3.wait()
        pltpu.sync_copy(ex_v, e_hbm.at[pl.ds(base, CH)])
        pltpu.sync_copy(oy_v, o_hbm.at[pl.ds(base, CH)])
        pltpu.sync_copy(bv_v, bb_hbm.at[pl.ds(base, CH)])

    return k(Wy, By_flat, X, Y)


def _tc_score(E, H0, O, Bb_row, W_ih, W_hh, b_ih, b_hh, tn=512):
    Bm, D = E.shape
    Bn = O.shape[0]
    grid = (Bn // tn,)

    def body(e_ref, h0_ref, o_ref, bb_ref, wih_ref, whh_ref, bih_ref, bhh_ref,
             out_ref, h_scr):
        j = pl.program_id(0)

        @pl.when(j == 0)
        def _():
            e = e_ref[...]
            h0 = h0_ref[...]
            gi = lax.dot_general(e, wih_ref[...], (((1,), (1,)), ((), ())),
                                 preferred_element_type=jnp.float32) + bih_ref[...]
            gh = lax.dot_general(h0, whh_ref[...], (((1,), (1,)), ((), ())),
                                 preferred_element_type=jnp.float32) + bhh_ref[...]
            r = jax.nn.sigmoid(gi[:, :D] + gh[:, :D])
            z = jax.nn.sigmoid(gi[:, D:2 * D] + gh[:, D:2 * D])
            n = jnp.tanh(gi[:, 2 * D:] + r * gh[:, 2 * D:])
            h_scr[...] = (1.0 - z) * n + z * h0

        out_ref[...] = lax.dot_general(
            h_scr[...], o_ref[...], (((1,), (1,)), ((), ())),
            preferred_element_type=jnp.float32) + bb_ref[...]

    return pl.pallas_call(
        body,
        grid=grid,
        in_specs=[
            pl.BlockSpec((Bm, D), lambda j: (0, 0)),
            pl.BlockSpec((Bm, D), lambda j: (0, 0)),
            pl.BlockSpec((tn, D), lambda j: (j, 0)),
            pl.BlockSpec((1, tn), lambda j: (0, j)),
            pl.BlockSpec((3 * D, D), lambda j: (0, 0)),
            pl.BlockSpec((3 * D, D), lambda j: (0, 0)),
            pl.BlockSpec((1, 3 * D), lambda j: (0, 0)),
            pl.BlockSpec((1, 3 * D), lambda j: (0, 0)),
        ],
        out_specs=pl.BlockSpec((Bm, tn), lambda j: (0, j)),
        out_shape=jax.ShapeDtypeStruct((Bm, Bn), jnp.float32),
        scratch_shapes=[pltpu.VMEM((Bm, D), jnp.float32)],
    )(E, H0, O, Bb_row, W_ih, W_hh, b_ih, b_hh)


def kernel(X, H, Y, Wy, By, W_ih, W_hh, b_ih, b_hh):
    X = X.astype(jnp.int32)
    Y = Y.astype(jnp.int32)
    E, O, Bb = _sc_gather(Wy, By.reshape(-1), X, Y)
    R = _tc_score(E, H[0], O, Bb.reshape(1, -1),
                  W_ih, W_hh, b_ih.reshape(1, -1), b_hh.reshape(1, -1))
    return R


# speedup vs baseline: 2.2998x; 1.0289x over previous
import functools

import jax
import jax.numpy as jnp
from jax import lax
from jax.experimental import pallas as pl
from jax.experimental.pallas import tpu as pltpu
from jax.experimental.pallas import tpu_sc as plsc


def _sc_gather(Wy, By_flat, X, Y):
    B = X.shape[0]
    D = Wy.shape[1]
    NW = 32
    CH = B // NW

    mesh = plsc.VectorSubcoreMesh(core_axis_name="c", subcore_axis_name="s")

    @functools.partial(
        pl.kernel,
        out_type=(
            jax.ShapeDtypeStruct((B, D), jnp.float32),
            jax.ShapeDtypeStruct((B, D), jnp.float32),
            jax.ShapeDtypeStruct((B,), jnp.float32),
        ),
        mesh=mesh,
        scratch_types=[
            pltpu.VMEM((2, CH), jnp.int32),
            pltpu.VMEM((CH, D), jnp.float32),
            pltpu.VMEM((CH, D), jnp.float32),
            pltpu.VMEM((CH,), jnp.float32),
            pltpu.SemaphoreType.DMA,
        ],
    )
    def k(wy_hbm, by_hbm, x_hbm, y_hbm, e_hbm, o_hbm, bb_hbm,
          idx_v, ex_v, oy_v, bv_v, sem):
        wid = lax.axis_index("s") * 2 + lax.axis_index("c")
        base = wid * CH
        pltpu.sync_copy(x_hbm.at[pl.ds(base, CH)], idx_v.at[0])
        pltpu.sync_copy(y_hbm.at[pl.ds(base, CH)], idx_v.at[1])
        c1 = pltpu.async_copy(wy_hbm.at[idx_v.at[0]], ex_v, sem)
        c2 = pltpu.async_copy(wy_hbm.at[idx_v.at[1]], oy_v, sem)
        c3 = pltpu.async_copy(by_hbm.at[idx_v.at[1]], bv_v, sem)
        c1.wait()
        c2.wait()
        c3.wait()
        pltpu.sync_copy(ex_v, e_hbm.at[pl.ds(base, CH)])
        pltpu.sync_copy(oy_v, o_hbm.at[pl.ds(base, CH)])
        pltpu.sync_copy(bv_v, bb_hbm.at[pl.ds(base, CH)])

    return k(Wy, By_flat, X, Y)


def _tc_score(E, H0, O, Bb_row, W_ih, W_hh, b_ih, b_hh, tm=512):
    Bm, D = E.shape
    Bn = O.shape[0]
    grid = (Bm // tm,)

    def body(e_ref, h0_ref, o_ref, bb_ref, wih_ref, whh_ref, bih_ref, bhh_ref,
             out_ref):
        e = e_ref[...]
        h0 = h0_ref[...]
        gi = lax.dot_general(e, wih_ref[...], (((1,), (1,)), ((), ())),
                             preferred_element_type=jnp.float32) + bih_ref[...]
        gh = lax.dot_general(h0, whh_ref[...], (((1,), (1,)), ((), ())),
                             preferred_element_type=jnp.float32) + bhh_ref[...]
        r = jax.nn.sigmoid(gi[:, :D] + gh[:, :D])
        z = jax.nn.sigmoid(gi[:, D:2 * D] + gh[:, D:2 * D])
        n = jnp.tanh(gi[:, 2 * D:] + r * gh[:, 2 * D:])
        h = (1.0 - z) * n + z * h0
        out_ref[...] = lax.dot_general(
            h, o_ref[...], (((1,), (1,)), ((), ())),
            preferred_element_type=jnp.float32) + bb_ref[...]

    return pl.pallas_call(
        body,
        grid=grid,
        in_specs=[
            pl.BlockSpec((tm, D), lambda i: (i, 0)),
            pl.BlockSpec((tm, D), lambda i: (i, 0)),
            pl.BlockSpec((Bn, D), lambda i: (0, 0)),
            pl.BlockSpec((1, Bn), lambda i: (0, 0)),
            pl.BlockSpec((3 * D, D), lambda i: (0, 0)),
            pl.BlockSpec((3 * D, D), lambda i: (0, 0)),
            pl.BlockSpec((1, 3 * D), lambda i: (0, 0)),
            pl.BlockSpec((1, 3 * D), lambda i: (0, 0)),
        ],
        out_specs=pl.BlockSpec((tm, Bn), lambda i: (i, 0)),
        out_shape=jax.ShapeDtypeStruct((Bm, Bn), jnp.float32),
    )(E, H0, O, Bb_row, W_ih, W_hh, b_ih, b_hh)


def kernel(X, H, Y, Wy, By, W_ih, W_hh, b_ih, b_hh):
    X = X.astype(jnp.int32)
    Y = Y.astype(jnp.int32)
    E, O, Bb = _sc_gather(Wy, By.reshape(-1), X, Y)
    R = _tc_score(E, H[0], O, Bb.reshape(1, -1),
                  W_ih, W_hh, b_ih.reshape(1, -1), b_hh.reshape(1, -1))
    return R
